# Initial kernel scaffold; baseline (speedup 1.0000x reference)
import jax
import jax.numpy as jnp
from jax.experimental import pallas as pl


def kernel(batch_x, W_mf1, b_mf1, W_a1, b_a1, W_a2, b_a2, tm_w, tm_b, cm_w1, cm_b1, cm_w2, cm_b2, proj_w, proj_b):
    raise NotImplementedError("write your pallas kernel here")


# speedup vs baseline: 74.0538x; 74.0538x over previous
import functools
import numpy as np
import jax
import jax.numpy as jnp
from jax.experimental import pallas as pl
from jax.experimental.pallas import tpu as pltpu

B, L, C, PRED, K = 32, 720, 862, 720, 20
F = L // 2 + 1


def _dft_mats():
    t = np.arange(L, dtype=np.int64)
    f = np.arange(F, dtype=np.int64)
    ph = (2.0 * np.pi / L) * ((f[:, None] * t[None, :]) % L).astype(np.float64)
    cos = np.cos(ph)
    sin = np.sin(ph)
    COS = cos.astype(np.float32)
    SINN = (-sin).astype(np.float32)
    alpha = np.where((f == 0) | (f == L // 2), 1.0, 2.0) / L
    ICOS = (cos * alpha[:, None]).T.astype(np.float32)
    ISIN = (-sin * alpha[:, None]).T.astype(np.float32)
    return COS, SINN, ICOS, ISIN


def _body(x_ref, cos_ref, sinn_ref, icos_ref, isin_ref,
          wmf1_ref, bmf1_ref, wa1h_ref, wa1x_ref, ba1_ref, wa2_ref, ba2_ref,
          tmw_ref, tmb_ref, cmw1t_ref, cmb1_ref, cmw2t_ref, cmb2_ref,
          projw_ref, projb_ref, norm_ref, pred_ref):
    x = x_ref[0]
    f32 = jnp.float32
    dot = functools.partial(jnp.dot, preferred_element_type=f32,
                            precision=jax.lax.Precision.HIGHEST)

    xr = dot(cos_ref[...], x)
    xi = dot(sinn_ref[...], x)
    mag = xr * xr + xi * xi

    cur = jnp.full((1, C), jnp.inf, dtype=f32)
    for _ in range(K):
        cur = jnp.max(jnp.where(mag < cur, mag, -1.0), axis=0, keepdims=True)
    m = jnp.where(mag >= cur, 1.0, 0.0).astype(f32)

    x_filt = dot(icos_ref[...], xr * m) + dot(isin_ref[...], xi * m)
    norm_ref[0] = x - x_filt

    h1 = jnp.maximum(dot(wmf1_ref[...], x_filt) + bmf1_ref[...], 0.0)
    h2 = jnp.maximum(
        dot(wa1h_ref[...], h1) + dot(wa1x_ref[...], x) + ba1_ref[...], 0.0
    )
    h3 = dot(wa2_ref[...], h2) + ba2_ref[...]

    x2 = h3 + jnp.maximum(dot(tmw_ref[...], h3) + tmb_ref[...], 0.0)

    z = jnp.maximum(dot(x2, cmw1t_ref[...]) + cmb1_ref[...], 0.0)
    z = dot(z, cmw2t_ref[...]) + cmb2_ref[...]
    x3 = x2 + z

    pred_ref[0] = dot(projw_ref[...], x3) + projb_ref[...]


def kernel(batch_x, W_mf1, b_mf1, W_a1, b_a1, W_a2, b_a2, tm_w, tm_b,
           cm_w1, cm_b1, cm_w2, cm_b2, proj_w, proj_b):
    COS, SINN, ICOS, ISIN = _dft_mats()
    cos = jnp.asarray(COS)
    sinn = jnp.asarray(SINN)
    icos = jnp.asarray(ICOS)
    isin = jnp.asarray(ISIN)

    wa1h = W_a1[:, :64]
    wa1x = W_a1[:, 64:]
    bmf1 = b_mf1[:, None]
    ba1 = b_a1[:, None]
    ba2 = b_a2[:, None]
    tmb = tm_b[:, None]
    cmb1 = cm_b1[None, :]
    cmb2 = cm_b2[None, :]
    projb = proj_b[:, None]
    cmw1t = cm_w1.T
    cmw2t = cm_w2.T

    def whole(a):
        nd = a.ndim
        return pl.BlockSpec(a.shape, lambda b, _n=nd: (0,) * _n)

    operands = (batch_x, cos, sinn, icos, isin, W_mf1, bmf1, wa1h, wa1x,
                ba1, W_a2, ba2, tm_w, tmb, cmw1t, cmb1, cmw2t, cmb2,
                proj_w, projb)
    in_specs = [pl.BlockSpec((1, L, C), lambda b: (b, 0, 0))]
    in_specs += [whole(a) for a in operands[1:]]

    norm, pred = pl.pallas_call(
        _body,
        grid=(B,),
        in_specs=in_specs,
        out_specs=[
            pl.BlockSpec((1, L, C), lambda b: (b, 0, 0)),
            pl.BlockSpec((1, PRED, C), lambda b: (b, 0, 0)),
        ],
        out_shape=[
            jax.ShapeDtypeStruct((B, L, C), jnp.float32),
            jax.ShapeDtypeStruct((B, PRED, C), jnp.float32),
        ],
        compiler_params=pltpu.CompilerParams(
            dimension_semantics=("arbitrary",),
        ),
    )(*operands)
    return norm, pred


# speedup vs baseline: 154.5300x; 2.0867x over previous
import functools
import numpy as np
import jax
import jax.numpy as jnp
from jax.experimental import pallas as pl
from jax.experimental.pallas import tpu as pltpu

B, L, C, PRED, K = 32, 720, 862, 720, 20
F = L // 2 + 1


def _dft_mats():
    t = np.arange(L, dtype=np.int64)
    f = np.arange(F, dtype=np.int64)
    ph = (2.0 * np.pi / L) * ((f[:, None] * t[None, :]) % L).astype(np.float64)
    cos = np.cos(ph)
    sin = np.sin(ph)
    COS = cos.astype(np.float32)
    SINN = (-sin).astype(np.float32)
    alpha = np.where((f == 0) | (f == L // 2), 1.0, 2.0) / L
    ICOS = (cos * alpha[:, None]).T.astype(np.float32)
    ISIN = (-sin * alpha[:, None]).T.astype(np.float32)
    return COS, SINN, ICOS, ISIN


def _body(x_ref, cos_ref, sinn_ref, icos_ref, isin_ref,
          wmf1_ref, bmf1_ref, wa1h_ref, wa1x_ref, ba1_ref, wa2_ref, ba2_ref,
          tmw_ref, tmb_ref, cmw1t_ref, cmb1_ref, cmw2t_ref, cmb2_ref,
          projw_ref, projb_ref, norm_ref, pred_ref):
    x = x_ref[0]
    f32 = jnp.float32
    hdot = functools.partial(jnp.dot, preferred_element_type=f32,
                             precision=jax.lax.Precision.HIGHEST)
    dot = functools.partial(jnp.dot, preferred_element_type=f32,
                            precision=jax.lax.Precision.DEFAULT)

    xr = hdot(cos_ref[...], x)
    xi = hdot(sinn_ref[...], x)
    mag = xr * xr + xi * xi

    cur = jnp.full((1, C), jnp.inf, dtype=f32)
    for _ in range(K):
        cur = jnp.max(jnp.where(mag < cur, mag, -1.0), axis=0, keepdims=True)
    m = jnp.where(mag >= cur, 1.0, 0.0).astype(f32)

    x_filt = dot(icos_ref[...], xr * m) + dot(isin_ref[...], xi * m)
    norm_ref[0] = x - x_filt

    h1 = jnp.maximum(dot(wmf1_ref[...], x_filt) + bmf1_ref[...], 0.0)
    h2 = jnp.maximum(
        dot(wa1h_ref[...], h1) + dot(wa1x_ref[...], x) + ba1_ref[...], 0.0
    )
    h3 = dot(wa2_ref[...], h2) + ba2_ref[...]

    x2 = h3 + jnp.maximum(dot(tmw_ref[...], h3) + tmb_ref[...], 0.0)

    z = jnp.maximum(dot(x2, cmw1t_ref[...]) + cmb1_ref[...], 0.0)
    z = dot(z, cmw2t_ref[...]) + cmb2_ref[...]
    x3 = x2 + z

    pred_ref[0] = dot(projw_ref[...], x3) + projb_ref[...]


def kernel(batch_x, W_mf1, b_mf1, W_a1, b_a1, W_a2, b_a2, tm_w, tm_b,
           cm_w1, cm_b1, cm_w2, cm_b2, proj_w, proj_b):
    COS, SINN, ICOS, ISIN = _dft_mats()
    cos = jnp.asarray(COS)
    sinn = jnp.asarray(SINN)
    icos = jnp.asarray(ICOS)
    isin = jnp.asarray(ISIN)

    wa1h = W_a1[:, :64]
    wa1x = W_a1[:, 64:]
    bmf1 = b_mf1[:, None]
    ba1 = b_a1[:, None]
    ba2 = b_a2[:, None]
    tmb = tm_b[:, None]
    cmb1 = cm_b1[None, :]
    cmb2 = cm_b2[None, :]
    projb = proj_b[:, None]
    cmw1t = cm_w1.T
    cmw2t = cm_w2.T

    def whole(a):
        nd = a.ndim
        return pl.BlockSpec(a.shape, lambda b, _n=nd: (0,) * _n)

    operands = (batch_x, cos, sinn, icos, isin, W_mf1, bmf1, wa1h, wa1x,
                ba1, W_a2, ba2, tm_w, tmb, cmw1t, cmb1, cmw2t, cmb2,
                proj_w, projb)
    in_specs = [pl.BlockSpec((1, L, C), lambda b: (b, 0, 0))]
    in_specs += [whole(a) for a in operands[1:]]

    norm, pred = pl.pallas_call(
        _body,
        grid=(B,),
        in_specs=in_specs,
        out_specs=[
            pl.BlockSpec((1, L, C), lambda b: (b, 0, 0)),
            pl.BlockSpec((1, PRED, C), lambda b: (b, 0, 0)),
        ],
        out_shape=[
            jax.ShapeDtypeStruct((B, L, C), jnp.float32),
            jax.ShapeDtypeStruct((B, PRED, C), jnp.float32),
        ],
        compiler_params=pltpu.CompilerParams(
            dimension_semantics=("arbitrary",),
        ),
    )(*operands)
    return norm, pred


# speedup vs baseline: 192.9309x; 1.2485x over previous
import functools
import numpy as np
import jax
import jax.numpy as jnp
from jax.experimental import pallas as pl
from jax.experimental.pallas import tpu as pltpu

B, L, C, PRED, K = 32, 720, 862, 720, 20
F = L // 2 + 1


def _dft_mats():
    t = np.arange(L, dtype=np.int64)
    f = np.arange(F, dtype=np.int64)
    ph = (2.0 * np.pi / L) * ((f[:, None] * t[None, :]) % L).astype(np.float64)
    cos = np.cos(ph)
    sin = np.sin(ph)
    COS = cos.astype(np.float32)
    SINN = (-sin).astype(np.float32)
    alpha = np.where((f == 0) | (f == L // 2), 1.0, 2.0) / L
    ICOS = (cos * alpha[:, None]).T.astype(np.float32)
    ISIN = (-sin * alpha[:, None]).T.astype(np.float32)
    return COS, SINN, ICOS, ISIN


def _body(x_ref, cos_hi_ref, cos_lo_ref, sinn_hi_ref, sinn_lo_ref,
          icos_ref, isin_ref,
          wmf1_ref, bmf1_ref, wa1h_ref, wa1x_ref, ba1_ref, wa2_ref, ba2_ref,
          tmw_ref, tmb_ref, cmw1t_ref, cmb1_ref, cmw2t_ref, cmb2_ref,
          projw_ref, projb_ref, norm_ref, pred_ref):
    x = x_ref[0]
    f32 = jnp.float32
    bf16 = jnp.bfloat16
    dot = functools.partial(jnp.dot, preferred_element_type=f32,
                            precision=jax.lax.Precision.DEFAULT)

    x_hi = x.astype(bf16)
    x_lo = (x - x_hi.astype(f32)).astype(bf16)
    xr = (dot(cos_hi_ref[...], x_hi)
          + (dot(cos_hi_ref[...], x_lo) + dot(cos_lo_ref[...], x_hi)))
    xi = (dot(sinn_hi_ref[...], x_hi)
          + (dot(sinn_hi_ref[...], x_lo) + dot(sinn_lo_ref[...], x_hi)))
    mag = xr * xr + xi * xi

    cur = jnp.full((1, C), jnp.inf, dtype=f32)
    for _ in range(K):
        cur = jnp.max(jnp.where(mag < cur, mag, -1.0), axis=0, keepdims=True)
    m = jnp.where(mag >= cur, 1.0, 0.0).astype(f32)

    x_filt = dot(icos_ref[...], xr * m) + dot(isin_ref[...], xi * m)
    norm_ref[0] = x - x_filt

    h1 = jnp.maximum(dot(wmf1_ref[...], x_filt) + bmf1_ref[...], 0.0)
    h2 = jnp.maximum(
        dot(wa1h_ref[...], h1) + dot(wa1x_ref[...], x) + ba1_ref[...], 0.0
    )
    h3 = dot(wa2_ref[...], h2) + ba2_ref[...]

    x2 = h3 + jnp.maximum(dot(tmw_ref[...], h3) + tmb_ref[...], 0.0)

    z = jnp.maximum(dot(x2, cmw1t_ref[...]) + cmb1_ref[...], 0.0)
    z = dot(z, cmw2t_ref[...]) + cmb2_ref[...]
    x3 = x2 + z

    pred_ref[0] = dot(projw_ref[...], x3) + projb_ref[...]


def kernel(batch_x, W_mf1, b_mf1, W_a1, b_a1, W_a2, b_a2, tm_w, tm_b,
           cm_w1, cm_b1, cm_w2, cm_b2, proj_w, proj_b):
    COS, SINN, ICOS, ISIN = _dft_mats()

    def split(m):
        hi = m.astype(np.float32).astype(jnp.bfloat16)
        lo = (m - np.asarray(hi).astype(np.float32)).astype(np.float32)
        return jnp.asarray(hi), jnp.asarray(lo).astype(jnp.bfloat16)

    cos_hi, cos_lo = split(COS)
    sinn_hi, sinn_lo = split(SINN)
    icos = jnp.asarray(ICOS)
    isin = jnp.asarray(ISIN)

    wa1h = W_a1[:, :64]
    wa1x = W_a1[:, 64:]
    bmf1 = b_mf1[:, None]
    ba1 = b_a1[:, None]
    ba2 = b_a2[:, None]
    tmb = tm_b[:, None]
    cmb1 = cm_b1[None, :]
    cmb2 = cm_b2[None, :]
    projb = proj_b[:, None]
    cmw1t = cm_w1.T
    cmw2t = cm_w2.T

    def whole(a):
        nd = a.ndim
        return pl.BlockSpec(a.shape, lambda b, _n=nd: (0,) * _n)

    operands = (batch_x, cos_hi, cos_lo, sinn_hi, sinn_lo, icos, isin,
                W_mf1, bmf1, wa1h, wa1x,
                ba1, W_a2, ba2, tm_w, tmb, cmw1t, cmb1, cmw2t, cmb2,
                proj_w, projb)
    in_specs = [pl.BlockSpec((1, L, C), lambda b: (b, 0, 0))]
    in_specs += [whole(a) for a in operands[1:]]

    norm, pred = pl.pallas_call(
        _body,
        grid=(B,),
        in_specs=in_specs,
        out_specs=[
            pl.BlockSpec((1, L, C), lambda b: (b, 0, 0)),
            pl.BlockSpec((1, PRED, C), lambda b: (b, 0, 0)),
        ],
        out_shape=[
            jax.ShapeDtypeStruct((B, L, C), jnp.float32),
            jax.ShapeDtypeStruct((B, PRED, C), jnp.float32),
        ],
        compiler_params=pltpu.CompilerParams(
            dimension_semantics=("arbitrary",),
        ),
    )(*operands)
    return norm, pred
